# baseline (speedup 1.0000x reference)
import jax
import jax.numpy as jnp
from jax.experimental import pallas as pl
from jax.experimental.pallas import tpu as pltpu

_R, _C = 160, 128
_P = _R * _C
_G = _R // 8
_MOUT = 200
_BIG = 2**30
_B = 8


def _tile_iota():
    return (jax.lax.broadcasted_iota(jnp.int32, (8, _C), 0) * _C
            + jax.lax.broadcasted_iota(jnp.int32, (8, _C), 1))


def _sel_tuple(take, b, a):
    return tuple(jnp.where(take, bx, ax) for bx, ax in zip(b, a))


def _combine(a, b):
    take_b = (b[0] > a[0]) | ((b[0] == a[0]) & (b[1] < a[1]))
    return _sel_tuple(take_b, b, a)


def _lex_chain(t):
    for axis, shifts in ((1, (1, 2, 4, 8, 16, 32, 64)), (0, (1, 2, 4))):
        for sh in shifts:
            r = tuple(pltpu.roll(x, sh, axis) for x in t)
            t = _combine(t, r)
    return t


def _or_bc(x):
    for axis, shifts in ((1, (1, 2, 4, 8, 16, 32, 64)), (0, (1, 2, 4))):
        for sh in shifts:
            x = jnp.maximum(x, pltpu.roll(x, sh, axis))
    return x


def _nms_kernel(thr_ref, x1, y1, x2, y2, s, sel_ref, num_ref, ws, ar):
    iou_thr = thr_ref[0, 0]
    score_thr = thr_ref[1, 0]
    ws[...] = jnp.where(s[...] > score_thr, s[...], -jnp.inf)
    ar[...] = (x2[...] - x1[...]) * (y2[...] - y1[...])

    ti = _tile_iota()

    def leaf(g, w_g):
        gs = pl.ds(g * 8, 8)
        return (w_g, ti + g * 1024, x1[gs, :], y1[gs, :], x2[gs, :],
                y2[gs, :], ar[gs, :])

    def tree_sweep(i_bc, valid_v):
        acc0 = acc1 = None
        for g in range(_G):
            w_g = ws[pl.ds(g * 8, 8), :]
            if i_bc is not None:
                pick_g = valid_v & ((ti + g * 1024) == i_bc)
                w_g = jnp.where(pick_g, -jnp.inf, w_g)
                ws[pl.ds(g * 8, 8), :] = w_g
            node = leaf(g, w_g)
            if g % 2 == 0:
                acc0 = node if acc0 is None else _combine(acc0, node)
            else:
                acc1 = node if acc1 is None else _combine(acc1, node)
        return _combine(acc0, acc1)

    top0 = tree_sweep(None, None)

    def round_fn(state):
        num_v, stopped_v, sel, sx1, sy1, sx2, sy2, sa, top = state
        m_bc, i_bc, b0, b1, b2, b3, a = _lex_chain(top)
        valid_v = (m_bc > -jnp.inf) & (stopped_v == 0)
        xx1 = jnp.maximum(b0, sx1)
        yy1 = jnp.maximum(b1, sy1)
        xx2 = jnp.minimum(b2, sx2)
        yy2 = jnp.minimum(b3, sy2)
        inter = (jnp.clip(xx2 - xx1, 0.0, None)
                 * jnp.clip(yy2 - yy1, 0.0, None))
        union = jnp.maximum(a + sa - inter, 1e-6)
        iou = inter / union
        hit = (iou >= iou_thr) & (ti < num_v)
        anyhit = _or_bc(hit.astype(jnp.int32))
        accepted = valid_v & (anyhit == 0) & (num_v < _MOUT)
        slot = accepted & (ti == num_v)
        sel = jnp.where(slot, i_bc, sel)
        sx1 = jnp.where(slot, b0, sx1)
        sy1 = jnp.where(slot, b1, sy1)
        sx2 = jnp.where(slot, b2, sx2)
        sy2 = jnp.where(slot, b3, sy2)
        sa = jnp.where(slot, a, sa)
        num_v = num_v + accepted.astype(jnp.int32)
        top = tree_sweep(i_bc, valid_v)
        stopped_v = jnp.maximum(stopped_v,
                                (m_bc == -jnp.inf).astype(jnp.int32))
        return (num_v, stopped_v, sel, sx1, sy1, sx2, sy2, sa, top)

    def cond(carry):
        num_s, stop_s = carry[0], carry[1]
        return jnp.logical_and(num_s < _MOUT, jnp.logical_not(stop_s))

    def body(carry):
        state = carry[2:10] + (carry[10:],)
        for _ in range(_B):
            state = round_fn(state)
        enc = state[0] + state[1] * 65536
        e = jnp.max(enc)
        return (jnp.bitwise_and(e, 65535), e >= 65536) + state[:8] + state[8]

    zf = jnp.zeros((8, _C), jnp.float32)
    zi = jnp.zeros((8, _C), jnp.int32)
    carry = (jnp.int32(0), jnp.bool_(False),
             zi, zi, zi, zf, zf, zf, zf, zf) + top0
    carry = jax.lax.while_loop(cond, body, carry)
    sel_ref[...] = carry[4]
    num_ref[0, 0] = carry[0]


def kernel(boxes, scores, max_output_size, iou_threshold, scores_threshold):
    boxes = boxes.astype(jnp.float32)
    scores = scores.astype(jnp.float32)
    n = boxes.shape[0]
    pad = _P - n
    bx = jnp.pad(boxes, ((0, pad), (0, 0)))
    planes = bx.T.reshape(4, _R, _C)
    s = jnp.pad(scores, (0, pad), constant_values=-jnp.inf).reshape(_R, _C)
    thr = jnp.stack([jnp.asarray(iou_threshold, jnp.float32),
                     jnp.asarray(scores_threshold, jnp.float32)]).reshape(2, 1)

    sel_m, num_m = pl.pallas_call(
        _nms_kernel,
        in_specs=[
            pl.BlockSpec(memory_space=pltpu.SMEM),
            pl.BlockSpec(memory_space=pltpu.VMEM),
            pl.BlockSpec(memory_space=pltpu.VMEM),
            pl.BlockSpec(memory_space=pltpu.VMEM),
            pl.BlockSpec(memory_space=pltpu.VMEM),
            pl.BlockSpec(memory_space=pltpu.VMEM),
        ],
        out_specs=[
            pl.BlockSpec(memory_space=pltpu.VMEM),
            pl.BlockSpec(memory_space=pltpu.SMEM),
        ],
        out_shape=[
            jax.ShapeDtypeStruct((8, _C), jnp.int32),
            jax.ShapeDtypeStruct((1, 1), jnp.int32),
        ],
        scratch_shapes=[
            pltpu.VMEM((_R, _C), jnp.float32),
            pltpu.VMEM((_R, _C), jnp.float32),
        ],
    )(thr, planes[0], planes[1], planes[2], planes[3], s)

    sel = sel_m.reshape(-1)[:_MOUT]
    num = jnp.minimum(num_m[0, 0], jnp.asarray(max_output_size, jnp.int32))
    return (sel, num)
